# baseline (speedup 1.0000x reference)
import numpy as np
import jax
import jax.numpy as jnp
from jax.experimental import pallas as pl
from jax.experimental.pallas import tpu as pltpu

_DET_LENGTH = 128
_REPS = 3
_CLIP = 20.0
_T = _REPS * _DET_LENGTH
_GM = np.array([[1, 0, 1, 1, 0, 1, 1], [1, 1, 1, 1, 0, 0, 1]], dtype=np.int64)


def _rotl6(p, k):
    return ((p << k) | (p >> (6 - k))) & 63


def _br0_signs():
    s = np.arange(64)
    j = s >> 1
    reg = np.stack([s & 1] + [(j >> i) & 1 for i in range(6)], axis=1)
    bits = (reg @ _GM.T) % 2
    return (1.0 - 2.0 * bits).astype(np.float32)


_BR0 = _br0_signs()

_S_PH = np.zeros((3 * 64, 2), np.float32)
_W_PH = np.zeros((3 * 4, 64), np.float32)
for _k in range(3):
    _p = np.arange(64)
    _s = _rotl6(_p, _k + 1)
    _S_PH[64 * _k + _p] = _BR0[_s]
    for _pp in range(64):
        _ss = int(_s[_pp])
        _W_PH[4 * _k + (_ss >> 4), _pp] = float(1 << (_ss & 15))

_BB = 256
_NSG = _T // 24
_SG0 = 5


def _viterbi_body(x0_ref, x1_ref, sp_ref, wp_ref, out_ref, dec_ref):
    bb = x0_ref.shape[2]
    Sph = sp_ref[...]
    Wph = wp_ref[...]
    dn = (((1,), (0,)), ((), ()))

    def _llr_rows(pg):
        x0 = jnp.clip(x0_ref[pg], -_CLIP, _CLIP).astype(jnp.bfloat16)
        x1 = jnp.clip(x1_ref[pg], -_CLIP, _CLIP).astype(jnp.bfloat16)
        return x0, x1

    def _expand(prob, k):
        if k == 0:
            p0 = jnp.concatenate([prob[0:32]] * 2, axis=0)
            p1 = jnp.concatenate([prob[32:64]] * 2, axis=0)
        elif k == 1:
            p0 = jnp.concatenate(
                [prob[0:16], prob[0:16], prob[32:48], prob[32:48]], axis=0
            )
            p1 = jnp.concatenate(
                [prob[16:32], prob[16:32], prob[48:64], prob[48:64]], axis=0
            )
        else:
            p0 = jnp.concatenate(
                [prob[8 * c : 8 * c + 8] for c in (0, 0, 2, 2, 4, 4, 6, 6)], axis=0
            )
            p1 = jnp.concatenate(
                [prob[8 * c : 8 * c + 8] for c in (1, 1, 3, 3, 5, 5, 7, 7)], axis=0
            )
        return p0, p1

    def _restore(q):
        return q.reshape(8, 8, bb).swapaxes(0, 1).reshape(64, bb)

    def _step(prob, cb, k):
        brA = jax.lax.dot_general(
            Sph[64 * k : 64 * k + 64], cb, dn, preferred_element_type=jnp.float32
        )
        p0, p1 = _expand(prob, k)
        c0 = p0 + brA
        c1 = p1 - brA
        nxt = jnp.maximum(c0, c1)
        prob = nxt - jnp.max(nxt, axis=0, keepdims=True)
        if k == 2:
            prob = _restore(prob)
        return prob, (c1 > c0)

    def _super(G, prob, pack):
        pairs = [
            _llr_rows(jax.lax.rem(3 * G + j, 16)) for j in range(3)
        ]
        words = []
        for u in range(24):
            j, i = divmod(u, 8)
            k = u % 3
            x0b, x1b = pairs[j]
            cb = jnp.concatenate([x0b[i : i + 1], x1b[i : i + 1]], axis=0)
            prob, dmask = _step(prob, cb, k)
            if pack:
                dec64 = jnp.where(dmask, jnp.float32(1), jnp.float32(0)).astype(
                    jnp.bfloat16
                )
                r = jax.lax.dot_general(
                    Wph[4 * k : 4 * k + 4], dec64, dn,
                    preferred_element_type=jnp.float32,
                )
                ri = r.astype(jnp.int32)
                words.append(ri[0:1] + (ri[1:2] << 16))
                words.append(ri[2:3] + (ri[3:4] << 16))
        if pack:
            dec_ref[G - _SG0] = jnp.concatenate(words, axis=0)
        return prob

    prob0 = jnp.zeros((64, bb), jnp.float32)
    prob0 = jax.lax.fori_loop(0, _SG0, lambda G, p: _super(G, p, False), prob0)
    jax.lax.fori_loop(_SG0, _NSG, lambda G, p: _super(G, p, True), prob0)

    def _step_back(state, w0, w1):
        w = jnp.where(state < 32, w0, w1)
        sh = state & 31
        d = (w >> sh) & 1
        return (state >> 1) + d * 32

    def tb_super(gi, state):
        G = _NSG - 1 - gi
        tile = dec_ref[G - _SG0]
        for j in (2, 1, 0):
            bits = [None] * 8
            for i in range(7, -1, -1):
                u = 8 * j + i
                bits[i] = ((state + 1) & 1).astype(jnp.float32)
                state = _step_back(
                    state, tile[2 * u : 2 * u + 1], tile[2 * u + 1 : 2 * u + 2]
                )
            blk = 3 * G + j
            bits8 = jnp.concatenate(bits, axis=0)

            @pl.when(jnp.logical_and(blk >= 16, blk < 32))
            def _(bits8=bits8, blk=blk):
                out_ref[blk - 16] = bits8
        return state

    state = jnp.zeros((1, bb), jnp.int32)
    jax.lax.fori_loop(0, _NSG - _SG0, tb_super, state)


def kernel(x):
    b = x.shape[0]
    bb = min(_BB, b)
    x0 = jnp.transpose(x[:, 0::2]).reshape(16, 8, b)
    x1 = jnp.transpose(x[:, 1::2]).reshape(16, 8, b)
    res = pl.pallas_call(
        _viterbi_body,
        out_shape=jax.ShapeDtypeStruct((16, 8, b), jnp.float32),
        grid=(b // bb,),
        in_specs=[
            pl.BlockSpec((16, 8, bb), lambda i: (0, 0, i)),
            pl.BlockSpec((16, 8, bb), lambda i: (0, 0, i)),
            pl.BlockSpec((192, 2), lambda i: (0, 0)),
            pl.BlockSpec((12, 64), lambda i: (0, 0)),
        ],
        out_specs=pl.BlockSpec((16, 8, bb), lambda i: (0, 0, i)),
        scratch_shapes=[pltpu.VMEM((_NSG - _SG0, 48, bb), jnp.int32)],
    )(
        x0,
        x1,
        jnp.asarray(_S_PH, dtype=jnp.bfloat16),
        jnp.asarray(_W_PH, dtype=jnp.bfloat16),
    )
    return res.reshape(_DET_LENGTH, b).transpose(1, 0)


# speedup vs baseline: 2.0001x; 2.0001x over previous
import numpy as np
import jax
import jax.numpy as jnp
from jax.experimental import pallas as pl
from jax.experimental.pallas import tpu as pltpu

_DET_LENGTH = 128
_REPS = 3
_CLIP = 20.0
_T = _REPS * _DET_LENGTH
_GM = np.array([[1, 0, 1, 1, 0, 1, 1], [1, 1, 1, 1, 0, 0, 1]], dtype=np.int64)


def _rotl6(p, k):
    return ((p << k) | (p >> (6 - k))) & 63


def _br0_signs():
    s = np.arange(64)
    j = s >> 1
    reg = np.stack([s & 1] + [(j >> i) & 1 for i in range(6)], axis=1)
    bits = (reg @ _GM.T) % 2
    return (1.0 - 2.0 * bits).astype(np.float32)


_BR0 = _br0_signs()

_S_PH = np.zeros((3 * 64, 2), np.float32)
_W_PH = np.zeros((3 * 4, 64), np.float32)
for _k in range(3):
    _p = np.arange(64)
    _s = _rotl6(_p, _k + 1)
    _S_PH[64 * _k + _p] = _BR0[_s]
    for _pp in range(64):
        _ss = int(_s[_pp])
        _W_PH[4 * _k + (_ss >> 4), _pp] = float(1 << (_ss & 15))

_BB = 1024
_NSG = _T // 24
_SG0 = 5


def _viterbi_body(x0_ref, x1_ref, sp_ref, wp_ref, out_ref, dec_ref):
    bb = x0_ref.shape[2]
    Sph = sp_ref[...]
    Wph = wp_ref[...]
    dn = (((1,), (0,)), ((), ()))

    def _llr_rows(pg):
        x0 = jnp.clip(x0_ref[pg], -_CLIP, _CLIP).astype(jnp.bfloat16)
        x1 = jnp.clip(x1_ref[pg], -_CLIP, _CLIP).astype(jnp.bfloat16)
        return x0, x1

    def _expand(prob, k):
        if k == 0:
            p0 = jnp.concatenate([prob[0:32]] * 2, axis=0)
            p1 = jnp.concatenate([prob[32:64]] * 2, axis=0)
        elif k == 1:
            p0 = jnp.concatenate(
                [prob[0:16], prob[0:16], prob[32:48], prob[32:48]], axis=0
            )
            p1 = jnp.concatenate(
                [prob[16:32], prob[16:32], prob[48:64], prob[48:64]], axis=0
            )
        else:
            p0 = jnp.concatenate(
                [prob[8 * c : 8 * c + 8] for c in (0, 0, 2, 2, 4, 4, 6, 6)], axis=0
            )
            p1 = jnp.concatenate(
                [prob[8 * c : 8 * c + 8] for c in (1, 1, 3, 3, 5, 5, 7, 7)], axis=0
            )
        return p0, p1

    def _restore(q):
        return q.reshape(8, 8, bb).swapaxes(0, 1).reshape(64, bb)

    def _step(prob, cb, k):
        brA = jax.lax.dot_general(
            Sph[64 * k : 64 * k + 64], cb, dn, preferred_element_type=jnp.float32
        )
        p0, p1 = _expand(prob, k)
        c0 = p0 + brA
        c1 = p1 - brA
        nxt = jnp.maximum(c0, c1)
        prob = nxt - jnp.max(nxt, axis=0, keepdims=True)
        if k == 2:
            prob = _restore(prob)
        return prob, (c1 > c0)

    def _super(G, prob, pack):
        pairs = [
            _llr_rows(jax.lax.rem(3 * G + j, 16)) for j in range(3)
        ]
        words = []
        for u in range(24):
            j, i = divmod(u, 8)
            k = u % 3
            x0b, x1b = pairs[j]
            cb = jnp.concatenate([x0b[i : i + 1], x1b[i : i + 1]], axis=0)
            prob, dmask = _step(prob, cb, k)
            if pack:
                dec64 = jnp.where(dmask, jnp.float32(1), jnp.float32(0)).astype(
                    jnp.bfloat16
                )
                r = jax.lax.dot_general(
                    Wph[4 * k : 4 * k + 4], dec64, dn,
                    preferred_element_type=jnp.float32,
                )
                ri = r.astype(jnp.int32)
                words.append(ri[0:1] + (ri[1:2] << 16))
                words.append(ri[2:3] + (ri[3:4] << 16))
        if pack:
            dec_ref[G - _SG0] = jnp.concatenate(words, axis=0)
        return prob

    prob0 = jnp.zeros((64, bb), jnp.float32)
    prob0 = jax.lax.fori_loop(0, _SG0, lambda G, p: _super(G, p, False), prob0)
    jax.lax.fori_loop(_SG0, _NSG, lambda G, p: _super(G, p, True), prob0)

    def _step_back(state, w0, w1):
        w = jnp.where(state < 32, w0, w1)
        sh = state & 31
        d = (w >> sh) & 1
        return (state >> 1) + d * 32

    def tb_super(gi, state):
        G = _NSG - 1 - gi
        tile = dec_ref[G - _SG0]
        for j in (2, 1, 0):
            bits = [None] * 8
            for i in range(7, -1, -1):
                u = 8 * j + i
                bits[i] = ((state + 1) & 1).astype(jnp.float32)
                state = _step_back(
                    state, tile[2 * u : 2 * u + 1], tile[2 * u + 1 : 2 * u + 2]
                )
            blk = 3 * G + j
            bits8 = jnp.concatenate(bits, axis=0)

            @pl.when(jnp.logical_and(blk >= 16, blk < 32))
            def _(bits8=bits8, blk=blk):
                out_ref[blk - 16] = bits8
        return state

    state = jnp.zeros((1, bb), jnp.int32)
    jax.lax.fori_loop(0, _NSG - _SG0, tb_super, state)


def kernel(x):
    b = x.shape[0]
    bb = min(_BB, b)
    x0 = jnp.transpose(x[:, 0::2]).reshape(16, 8, b)
    x1 = jnp.transpose(x[:, 1::2]).reshape(16, 8, b)
    res = pl.pallas_call(
        _viterbi_body,
        out_shape=jax.ShapeDtypeStruct((16, 8, b), jnp.float32),
        grid=(b // bb,),
        in_specs=[
            pl.BlockSpec((16, 8, bb), lambda i: (0, 0, i)),
            pl.BlockSpec((16, 8, bb), lambda i: (0, 0, i)),
            pl.BlockSpec((192, 2), lambda i: (0, 0)),
            pl.BlockSpec((12, 64), lambda i: (0, 0)),
        ],
        out_specs=pl.BlockSpec((16, 8, bb), lambda i: (0, 0, i)),
        scratch_shapes=[pltpu.VMEM((_NSG - _SG0, 48, bb), jnp.int32)],
    )(
        x0,
        x1,
        jnp.asarray(_S_PH, dtype=jnp.bfloat16),
        jnp.asarray(_W_PH, dtype=jnp.bfloat16),
    )
    return res.reshape(_DET_LENGTH, b).transpose(1, 0)


# speedup vs baseline: 2.3374x; 1.1687x over previous
import numpy as np
import jax
import jax.numpy as jnp
from jax.experimental import pallas as pl
from jax.experimental.pallas import tpu as pltpu

_DET_LENGTH = 128
_REPS = 3
_CLIP = 20.0
_T = _REPS * _DET_LENGTH
_GM = np.array([[1, 0, 1, 1, 0, 1, 1], [1, 1, 1, 1, 0, 0, 1]], dtype=np.int64)


def _rotl6(p, k):
    return ((p << k) | (p >> (6 - k))) & 63


def _br0_signs():
    s = np.arange(64)
    j = s >> 1
    reg = np.stack([s & 1] + [(j >> i) & 1 for i in range(6)], axis=1)
    bits = (reg @ _GM.T) % 2
    return (1.0 - 2.0 * bits).astype(np.float32)


_BR0 = _br0_signs()

_S_PH = np.zeros((3 * 64, 2), np.float32)
_W_PH = np.zeros((3 * 4, 64), np.float32)
for _k in range(3):
    _p = np.arange(64)
    _s = _rotl6(_p, _k + 1)
    _S_PH[64 * _k + _p] = _BR0[_s]
    for _pp in range(64):
        _ss = int(_s[_pp])
        _W_PH[4 * _k + (_ss >> 4), _pp] = float(1 << (_ss & 15))

_BB = 2048
_NSG = _T // 24
_SG0 = 5


def _viterbi_body(x0_ref, x1_ref, sp_ref, wp_ref, out_ref, dec_ref):
    bb = x0_ref.shape[2]
    Sph = sp_ref[...]
    Wph = wp_ref[...]
    dn = (((1,), (0,)), ((), ()))

    def _llr_rows(pg):
        x0 = jnp.clip(x0_ref[pg], -_CLIP, _CLIP).astype(jnp.bfloat16)
        x1 = jnp.clip(x1_ref[pg], -_CLIP, _CLIP).astype(jnp.bfloat16)
        return x0, x1

    def _expand(prob, k):
        if k == 0:
            p0 = jnp.concatenate([prob[0:32]] * 2, axis=0)
            p1 = jnp.concatenate([prob[32:64]] * 2, axis=0)
        elif k == 1:
            p0 = jnp.concatenate(
                [prob[0:16], prob[0:16], prob[32:48], prob[32:48]], axis=0
            )
            p1 = jnp.concatenate(
                [prob[16:32], prob[16:32], prob[48:64], prob[48:64]], axis=0
            )
        else:
            p0 = jnp.concatenate(
                [prob[8 * c : 8 * c + 8] for c in (0, 0, 2, 2, 4, 4, 6, 6)], axis=0
            )
            p1 = jnp.concatenate(
                [prob[8 * c : 8 * c + 8] for c in (1, 1, 3, 3, 5, 5, 7, 7)], axis=0
            )
        return p0, p1

    def _restore(q):
        return q.reshape(8, 8, bb).swapaxes(0, 1).reshape(64, bb)

    def _step(prob, cb, k):
        brA = jax.lax.dot_general(
            Sph[64 * k : 64 * k + 64], cb, dn, preferred_element_type=jnp.float32
        )
        p0, p1 = _expand(prob, k)
        c0 = p0 + brA
        c1 = p1 - brA
        nxt = jnp.maximum(c0, c1)
        prob = nxt - jnp.max(nxt, axis=0, keepdims=True)
        if k == 2:
            prob = _restore(prob)
        return prob, (c1 > c0)

    def _super(G, prob, pack):
        pairs = [
            _llr_rows(jax.lax.rem(3 * G + j, 16)) for j in range(3)
        ]
        words = []
        for u in range(24):
            j, i = divmod(u, 8)
            k = u % 3
            x0b, x1b = pairs[j]
            cb = jnp.concatenate([x0b[i : i + 1], x1b[i : i + 1]], axis=0)
            prob, dmask = _step(prob, cb, k)
            if pack:
                dec64 = jnp.where(dmask, jnp.float32(1), jnp.float32(0)).astype(
                    jnp.bfloat16
                )
                r = jax.lax.dot_general(
                    Wph[4 * k : 4 * k + 4], dec64, dn,
                    preferred_element_type=jnp.float32,
                )
                ri = r.astype(jnp.int32)
                words.append(ri[0:1] + (ri[1:2] << 16))
                words.append(ri[2:3] + (ri[3:4] << 16))
        if pack:
            dec_ref[G - _SG0] = jnp.concatenate(words, axis=0)
        return prob

    prob0 = jnp.zeros((64, bb), jnp.float32)
    prob0 = jax.lax.fori_loop(0, _SG0, lambda G, p: _super(G, p, False), prob0)
    jax.lax.fori_loop(_SG0, _NSG, lambda G, p: _super(G, p, True), prob0)

    def _step_back(state, w0, w1):
        w = jnp.where(state < 32, w0, w1)
        sh = state & 31
        d = (w >> sh) & 1
        return (state >> 1) + d * 32

    def tb_super(gi, state):
        G = _NSG - 1 - gi
        tile = dec_ref[G - _SG0]
        for j in (2, 1, 0):
            bits = [None] * 8
            for i in range(7, -1, -1):
                u = 8 * j + i
                bits[i] = ((state + 1) & 1).astype(jnp.float32)
                state = _step_back(
                    state, tile[2 * u : 2 * u + 1], tile[2 * u + 1 : 2 * u + 2]
                )
            blk = 3 * G + j
            bits8 = jnp.concatenate(bits, axis=0)

            @pl.when(jnp.logical_and(blk >= 16, blk < 32))
            def _(bits8=bits8, blk=blk):
                out_ref[blk - 16] = bits8
        return state

    state = jnp.zeros((1, bb), jnp.int32)
    jax.lax.fori_loop(0, _NSG - _SG0, tb_super, state)


def kernel(x):
    b = x.shape[0]
    bb = min(_BB, b)
    x0 = jnp.transpose(x[:, 0::2]).reshape(16, 8, b)
    x1 = jnp.transpose(x[:, 1::2]).reshape(16, 8, b)
    res = pl.pallas_call(
        _viterbi_body,
        out_shape=jax.ShapeDtypeStruct((16, 8, b), jnp.float32),
        grid=(b // bb,),
        in_specs=[
            pl.BlockSpec((16, 8, bb), lambda i: (0, 0, i)),
            pl.BlockSpec((16, 8, bb), lambda i: (0, 0, i)),
            pl.BlockSpec((192, 2), lambda i: (0, 0)),
            pl.BlockSpec((12, 64), lambda i: (0, 0)),
        ],
        out_specs=pl.BlockSpec((16, 8, bb), lambda i: (0, 0, i)),
        scratch_shapes=[pltpu.VMEM((_NSG - _SG0, 48, bb), jnp.int32)],
    )(
        x0,
        x1,
        jnp.asarray(_S_PH, dtype=jnp.bfloat16),
        jnp.asarray(_W_PH, dtype=jnp.bfloat16),
    )
    return res.reshape(_DET_LENGTH, b).transpose(1, 0)


# speedup vs baseline: 2.4633x; 1.0539x over previous
import numpy as np
import jax
import jax.numpy as jnp
from jax.experimental import pallas as pl
from jax.experimental.pallas import tpu as pltpu

_DET_LENGTH = 128
_REPS = 3
_CLIP = 20.0
_T = _REPS * _DET_LENGTH
_GM = np.array([[1, 0, 1, 1, 0, 1, 1], [1, 1, 1, 1, 0, 0, 1]], dtype=np.int64)


def _rotl6(p, k):
    return ((p << k) | (p >> (6 - k))) & 63


def _br0_signs():
    s = np.arange(64)
    j = s >> 1
    reg = np.stack([s & 1] + [(j >> i) & 1 for i in range(6)], axis=1)
    bits = (reg @ _GM.T) % 2
    return (1.0 - 2.0 * bits).astype(np.float32)


_BR0 = _br0_signs()

_S_PH = np.zeros((3 * 64, 2), np.float32)
_W_PH = np.zeros((3 * 4, 64), np.float32)
for _k in range(3):
    _p = np.arange(64)
    _s = _rotl6(_p, _k + 1)
    _S_PH[64 * _k + _p] = _BR0[_s]
    for _pp in range(64):
        _ss = int(_s[_pp])
        _W_PH[4 * _k + (_ss >> 4), _pp] = float(1 << (_ss & 15))

_BB = 4096
_NSG = _T // 24
_SG0 = 5


def _viterbi_body(x0_ref, x1_ref, sp_ref, wp_ref, out_ref, dec_ref):
    bb = x0_ref.shape[2]
    Sph = sp_ref[...]
    Wph = wp_ref[...]
    dn = (((1,), (0,)), ((), ()))

    def _llr_rows(pg):
        x0 = jnp.clip(x0_ref[pg], -_CLIP, _CLIP).astype(jnp.bfloat16)
        x1 = jnp.clip(x1_ref[pg], -_CLIP, _CLIP).astype(jnp.bfloat16)
        return x0, x1

    def _expand(prob, k):
        if k == 0:
            p0 = jnp.concatenate([prob[0:32]] * 2, axis=0)
            p1 = jnp.concatenate([prob[32:64]] * 2, axis=0)
        elif k == 1:
            p0 = jnp.concatenate(
                [prob[0:16], prob[0:16], prob[32:48], prob[32:48]], axis=0
            )
            p1 = jnp.concatenate(
                [prob[16:32], prob[16:32], prob[48:64], prob[48:64]], axis=0
            )
        else:
            p0 = jnp.concatenate(
                [prob[8 * c : 8 * c + 8] for c in (0, 0, 2, 2, 4, 4, 6, 6)], axis=0
            )
            p1 = jnp.concatenate(
                [prob[8 * c : 8 * c + 8] for c in (1, 1, 3, 3, 5, 5, 7, 7)], axis=0
            )
        return p0, p1

    def _restore(q):
        return q.reshape(8, 8, bb).swapaxes(0, 1).reshape(64, bb)

    def _step(prob, cb, k):
        brA = jax.lax.dot_general(
            Sph[64 * k : 64 * k + 64], cb, dn, preferred_element_type=jnp.float32
        )
        p0, p1 = _expand(prob, k)
        c0 = p0 + brA
        c1 = p1 - brA
        nxt = jnp.maximum(c0, c1)
        prob = nxt - jnp.max(nxt, axis=0, keepdims=True)
        if k == 2:
            prob = _restore(prob)
        return prob, (c1 > c0)

    def _super(G, prob, pack):
        pairs = [
            _llr_rows(jax.lax.rem(3 * G + j, 16)) for j in range(3)
        ]
        words = []
        for u in range(24):
            j, i = divmod(u, 8)
            k = u % 3
            x0b, x1b = pairs[j]
            cb = jnp.concatenate([x0b[i : i + 1], x1b[i : i + 1]], axis=0)
            prob, dmask = _step(prob, cb, k)
            if pack:
                dec64 = jnp.where(dmask, jnp.float32(1), jnp.float32(0)).astype(
                    jnp.bfloat16
                )
                r = jax.lax.dot_general(
                    Wph[4 * k : 4 * k + 4], dec64, dn,
                    preferred_element_type=jnp.float32,
                )
                ri = r.astype(jnp.int32)
                words.append(ri[0:1] + (ri[1:2] << 16))
                words.append(ri[2:3] + (ri[3:4] << 16))
        if pack:
            dec_ref[G - _SG0] = jnp.concatenate(words, axis=0)
        return prob

    prob0 = jnp.zeros((64, bb), jnp.float32)
    prob0 = jax.lax.fori_loop(0, _SG0, lambda G, p: _super(G, p, False), prob0)
    jax.lax.fori_loop(_SG0, _NSG, lambda G, p: _super(G, p, True), prob0)

    def _step_back(state, w0, w1):
        w = jnp.where(state < 32, w0, w1)
        sh = state & 31
        d = (w >> sh) & 1
        return (state >> 1) + d * 32

    def tb_super(gi, state):
        G = _NSG - 1 - gi
        tile = dec_ref[G - _SG0]
        for j in (2, 1, 0):
            bits = [None] * 8
            for i in range(7, -1, -1):
                u = 8 * j + i
                bits[i] = ((state + 1) & 1).astype(jnp.float32)
                state = _step_back(
                    state, tile[2 * u : 2 * u + 1], tile[2 * u + 1 : 2 * u + 2]
                )
            blk = 3 * G + j
            bits8 = jnp.concatenate(bits, axis=0)

            @pl.when(jnp.logical_and(blk >= 16, blk < 32))
            def _(bits8=bits8, blk=blk):
                out_ref[blk - 16] = bits8
        return state

    state = jnp.zeros((1, bb), jnp.int32)
    jax.lax.fori_loop(0, _NSG - _SG0, tb_super, state)


def kernel(x):
    b = x.shape[0]
    bb = min(_BB, b)
    x0 = jnp.transpose(x[:, 0::2]).reshape(16, 8, b)
    x1 = jnp.transpose(x[:, 1::2]).reshape(16, 8, b)
    res = pl.pallas_call(
        _viterbi_body,
        out_shape=jax.ShapeDtypeStruct((16, 8, b), jnp.float32),
        grid=(b // bb,),
        in_specs=[
            pl.BlockSpec((16, 8, bb), lambda i: (0, 0, i)),
            pl.BlockSpec((16, 8, bb), lambda i: (0, 0, i)),
            pl.BlockSpec((192, 2), lambda i: (0, 0)),
            pl.BlockSpec((12, 64), lambda i: (0, 0)),
        ],
        out_specs=pl.BlockSpec((16, 8, bb), lambda i: (0, 0, i)),
        scratch_shapes=[pltpu.VMEM((_NSG - _SG0, 48, bb), jnp.int32)],
    )(
        x0,
        x1,
        jnp.asarray(_S_PH, dtype=jnp.bfloat16),
        jnp.asarray(_W_PH, dtype=jnp.bfloat16),
    )
    return res.reshape(_DET_LENGTH, b).transpose(1, 0)
